# baseline (speedup 1.0000x reference)
import functools

import jax
import jax.numpy as jnp
from jax import lax
from jax.experimental import pallas as pl
from jax.experimental.pallas import tpu as pltpu
from jax.experimental.pallas import tpu_sc as plsc

B, C, OV, W, H = 128, 3, 4, 64, 64
HHALF = H // 2
NC, NS = 2, 16
NW = NC * NS
UNITS_PER_W = C * W // NW
LANES = 16
GRPS = HHALF * B // LANES
TAB_PAD = 352


def _unit_coords(u):
  c = lax.shift_right_logical(u, 6)
  w = lax.bitwise_and(u, 63)
  ch = jnp.where(c == 2, 0, c + 1)
  t = jnp.where(c == 1, 1, 0)
  return c, w, ch, t


def _sc_body(ids_hbm, vals_hbm, tab_hbm, dw_hbm, out_hbm,
             tab_v, dw_v, ids_v, vals_v, out_v,
             sem_i0, sem_i1, sem_v0, sem_v1, sem_o0, sem_o1):
  cid = lax.axis_index("c")
  sid = lax.axis_index("s")
  wid = sid * NC + cid

  sem_i = (sem_i0, sem_i1)
  sem_v = (sem_v0, sem_v1)
  sem_o = (sem_o0, sem_o1)

  def start_in(ch, w, half, buf):
    h0 = half * HHALF
    pltpu.async_copy(ids_hbm.at[ch, :, w, pl.ds(h0, HHALF), :], ids_v.at[buf],
                     sem_i[buf])
    pltpu.async_copy(vals_hbm.at[ch, :, w, pl.ds(h0, HHALF), :],
                     vals_v.at[buf], sem_v[buf])

  def wait_in(buf):
    pltpu.make_async_copy(ids_hbm.at[0, :, 0, pl.ds(0, HHALF), :],
                          ids_v.at[buf], sem_i[buf]).wait()
    pltpu.make_async_copy(vals_hbm.at[0, :, 0, pl.ds(0, HHALF), :],
                          vals_v.at[buf], sem_v[buf]).wait()

  def wait_out(buf):
    pltpu.make_async_copy(out_v.at[buf], out_hbm.at[0, 0, pl.ds(0, HHALF), :],
                          sem_o[buf]).wait()

  def run_stage(c, w, half, t, buf, first):
    wait_in(buf)
    if not first:
      wait_out(buf)
    t16 = jnp.full((LANES,), t, jnp.int32)
    dw = dw_v[pl.ds(t * LANES, LANES)]

    @plsc.parallel_loop(0, GRPS, unroll=2)
    def _compute(i):
      h = i >> 3
      b0 = (i & 7) * LANES
      idx = ids_v[buf, 0, h, pl.ds(b0, LANES)]
      acc = (plsc.load_gather(tab_v, [t16, idx])
             * vals_v[buf, 0, h, pl.ds(b0, LANES)])
      for ov in range(1, OV):
        idx = ids_v[buf, ov, h, pl.ds(b0, LANES)]
        acc = jnp.maximum(
            acc, plsc.load_gather(tab_v, [t16, idx])
            * vals_v[buf, ov, h, pl.ds(b0, LANES)])
      out_v[buf, h, pl.ds(b0, LANES)] = acc * dw

    h0 = half * HHALF
    pltpu.async_copy(out_v.at[buf], out_hbm.at[c, w, pl.ds(h0, HHALF), :],
                     sem_o[buf])

  def unit(q, carry):
    c, w, ch, t = _unit_coords(wid * UNITS_PER_W + q)
    start_in(ch, w, 1, 1)
    run_stage(c, w, 0, t, 0, False)

    @pl.when(q < UNITS_PER_W - 1)
    def _prefetch():
      _, w1, ch1, _ = _unit_coords(wid * UNITS_PER_W + q + 1)
      start_in(ch1, w1, 0, 0)

    run_stage(c, w, 1, t, 1, False)
    return carry

  c0, w0, ch0, t0 = _unit_coords(wid * UNITS_PER_W)
  start_in(ch0, w0, 0, 0)
  start_in(ch0, w0, 1, 1)
  pltpu.sync_copy(tab_hbm, tab_v)
  pltpu.sync_copy(dw_hbm, dw_v)
  run_stage(c0, w0, 0, t0, 0, True)
  _, w1, ch1, _ = _unit_coords(wid * UNITS_PER_W + 1)
  start_in(ch1, w1, 0, 0)
  run_stage(c0, w0, 1, t0, 1, True)
  lax.fori_loop(1, UNITS_PER_W, unit, 0)

  wait_out(0)
  wait_out(1)


@functools.cache
def _build_sc_call():
  return pl.kernel(
    _sc_body,
    out_type=jax.ShapeDtypeStruct((C, W, H, B), jnp.float32),
    mesh=plsc.VectorSubcoreMesh(
        core_axis_name="c", subcore_axis_name="s",
        num_cores=NC, num_subcores=NS),
    scratch_types=[
        pltpu.VMEM((2, TAB_PAD), jnp.float32),
        pltpu.VMEM((2 * LANES,), jnp.float32),
        pltpu.VMEM((2, OV, HHALF, B), jnp.int32),
        pltpu.VMEM((2, OV, HHALF, B), jnp.float32),
        pltpu.VMEM((2, HHALF, B), jnp.float32),
        pltpu.SemaphoreType.DMA,
        pltpu.SemaphoreType.DMA,
        pltpu.SemaphoreType.DMA,
        pltpu.SemaphoreType.DMA,
        pltpu.SemaphoreType.DMA,
        pltpu.SemaphoreType.DMA,
    ],
    compiler_params=pltpu.CompilerParams(
        needs_layout_passes=False, use_tc_tiling_on_sc=True),
  )


@jax.jit
def kernel(bag_of_units_ids, bag_of_units_values, player_embed, neutral_embed,
           player_dense_weight, neutral_dense_weight):
  ids = jnp.transpose(bag_of_units_ids, (1, 2, 3, 4, 0))
  vals = jnp.transpose(bag_of_units_values, (1, 2, 3, 4, 0))
  tab = jnp.stack([
      jnp.pad(player_embed.reshape(-1), (0, TAB_PAD - player_embed.size)),
      jnp.pad(neutral_embed.reshape(-1), (0, TAB_PAD - neutral_embed.size)),
  ])
  dw = jnp.concatenate([
      jnp.broadcast_to(player_dense_weight, (LANES,)),
      jnp.broadcast_to(neutral_dense_weight, (LANES,)),
  ])
  out = _build_sc_call()(ids, vals, tab, dw)
  return jnp.transpose(out, (3, 0, 1, 2))


# speedup vs baseline: 1.0019x; 1.0019x over previous
import functools

import jax
import jax.numpy as jnp
from jax import lax
from jax.experimental import pallas as pl
from jax.experimental.pallas import tpu as pltpu
from jax.experimental.pallas import tpu_sc as plsc

B, C, OV, W, H = 128, 3, 4, 64, 64
HHALF = H // 2
NC, NS = 2, 16
NW = NC * NS
UNITS_PER_W = C * W // NW
LANES = 16
GRPS = HHALF * B // LANES
TAB_PAD = 352


def _unit_coords(u):
  c = lax.shift_right_logical(u, 6)
  w = lax.bitwise_and(u, 63)
  ch = jnp.where(c == 2, 0, c + 1)
  t = jnp.where(c == 1, 1, 0)
  return c, w, ch, t


def _sc_body(ids_hbm, vals_hbm, tab_hbm, dw_hbm, out_hbm,
             tab_v, dw_v, ids_v, vals_v, out_v,
             sem_i0, sem_i1, sem_v0, sem_v1, sem_o0, sem_o1):
  cid = lax.axis_index("c")
  sid = lax.axis_index("s")
  wid = sid * NC + cid

  sem_i = (sem_i0, sem_i1)
  sem_v = (sem_v0, sem_v1)
  sem_o = (sem_o0, sem_o1)

  def start_in(ch, w, half, buf):
    h0 = half * HHALF
    pltpu.async_copy(ids_hbm.at[ch, :, w, pl.ds(h0, HHALF), :], ids_v.at[buf],
                     sem_i[buf])
    pltpu.async_copy(vals_hbm.at[ch, :, w, pl.ds(h0, HHALF), :],
                     vals_v.at[buf], sem_v[buf])

  def wait_in(buf):
    pltpu.make_async_copy(ids_hbm.at[0, :, 0, pl.ds(0, HHALF), :],
                          ids_v.at[buf], sem_i[buf]).wait()
    pltpu.make_async_copy(vals_hbm.at[0, :, 0, pl.ds(0, HHALF), :],
                          vals_v.at[buf], sem_v[buf]).wait()

  def wait_out(buf):
    pltpu.make_async_copy(out_v.at[buf], out_hbm.at[0, 0, pl.ds(0, HHALF), :],
                          sem_o[buf]).wait()

  def run_stage(c, w, half, t, buf, first):
    wait_in(buf)
    if not first:
      wait_out(buf)
    t16 = jnp.full((LANES,), t, jnp.int32)
    dw = dw_v[pl.ds(t * LANES, LANES)]

    @plsc.parallel_loop(0, GRPS, unroll=4)
    def _compute(i):
      h = i >> 3
      b0 = (i & 7) * LANES
      idx = ids_v[buf, 0, h, pl.ds(b0, LANES)]
      acc = (plsc.load_gather(tab_v, [t16, idx])
             * vals_v[buf, 0, h, pl.ds(b0, LANES)])
      for ov in range(1, OV):
        idx = ids_v[buf, ov, h, pl.ds(b0, LANES)]
        acc = jnp.maximum(
            acc, plsc.load_gather(tab_v, [t16, idx])
            * vals_v[buf, ov, h, pl.ds(b0, LANES)])
      out_v[buf, h, pl.ds(b0, LANES)] = acc * dw

    h0 = half * HHALF
    pltpu.async_copy(out_v.at[buf], out_hbm.at[c, w, pl.ds(h0, HHALF), :],
                     sem_o[buf])

  def unit(q, carry):
    c, w, ch, t = _unit_coords(wid * UNITS_PER_W + q)
    start_in(ch, w, 1, 1)
    run_stage(c, w, 0, t, 0, False)

    @pl.when(q < UNITS_PER_W - 1)
    def _prefetch():
      _, w1, ch1, _ = _unit_coords(wid * UNITS_PER_W + q + 1)
      start_in(ch1, w1, 0, 0)

    run_stage(c, w, 1, t, 1, False)
    return carry

  c0, w0, ch0, t0 = _unit_coords(wid * UNITS_PER_W)
  start_in(ch0, w0, 0, 0)
  start_in(ch0, w0, 1, 1)
  pltpu.sync_copy(tab_hbm, tab_v)
  pltpu.sync_copy(dw_hbm, dw_v)
  run_stage(c0, w0, 0, t0, 0, True)
  _, w1, ch1, _ = _unit_coords(wid * UNITS_PER_W + 1)
  start_in(ch1, w1, 0, 0)
  run_stage(c0, w0, 1, t0, 1, True)
  lax.fori_loop(1, UNITS_PER_W, unit, 0)

  wait_out(0)
  wait_out(1)


@functools.cache
def _build_sc_call():
  return pl.kernel(
    _sc_body,
    out_type=jax.ShapeDtypeStruct((C, W, H, B), jnp.float32),
    mesh=plsc.VectorSubcoreMesh(
        core_axis_name="c", subcore_axis_name="s",
        num_cores=NC, num_subcores=NS),
    scratch_types=[
        pltpu.VMEM((2, TAB_PAD), jnp.float32),
        pltpu.VMEM((2 * LANES,), jnp.float32),
        pltpu.VMEM((2, OV, HHALF, B), jnp.int32),
        pltpu.VMEM((2, OV, HHALF, B), jnp.float32),
        pltpu.VMEM((2, HHALF, B), jnp.float32),
        pltpu.SemaphoreType.DMA,
        pltpu.SemaphoreType.DMA,
        pltpu.SemaphoreType.DMA,
        pltpu.SemaphoreType.DMA,
        pltpu.SemaphoreType.DMA,
        pltpu.SemaphoreType.DMA,
    ],
    compiler_params=pltpu.CompilerParams(
        needs_layout_passes=False, use_tc_tiling_on_sc=True),
  )


@jax.jit
def kernel(bag_of_units_ids, bag_of_units_values, player_embed, neutral_embed,
           player_dense_weight, neutral_dense_weight):
  ids = jnp.transpose(bag_of_units_ids, (1, 2, 3, 4, 0))
  vals = jnp.transpose(bag_of_units_values, (1, 2, 3, 4, 0))
  tab = jnp.stack([
      jnp.pad(player_embed.reshape(-1), (0, TAB_PAD - player_embed.size)),
      jnp.pad(neutral_embed.reshape(-1), (0, TAB_PAD - neutral_embed.size)),
  ])
  dw = jnp.concatenate([
      jnp.broadcast_to(player_dense_weight, (LANES,)),
      jnp.broadcast_to(neutral_dense_weight, (LANES,)),
  ])
  out = _build_sc_call()(ids, vals, tab, dw)
  return jnp.transpose(out, (3, 0, 1, 2))
